# baseline (speedup 1.0000x reference)
import functools

import jax
import jax.numpy as jnp
from jax import lax
from jax.experimental import pallas as pl
from jax.experimental.pallas import tpu as pltpu
from jax.experimental.pallas import tpu_sc as plsc

L = 16
NC = 2
NS = 16
NW = NC * NS
T = 8
D = 16
N_TOK = 4 * 8192
CHUNK = N_TOK // NW
BLOCKS = CHUNK // L
NPAR = 5 + 2 * T


def _full(v, dtype=jnp.int32):
    return jnp.full((L,), v, dtype)


def _bf16_round(v):
    y = plsc.bitcast(v, jnp.int32)
    odd = lax.shift_right_logical(y, 16) & 1
    r = (y + 32767 + odd) & (-65536)
    return plsc.bitcast(r, jnp.float32)


def _sc_router_body(x_hbm, pos_hbm, par_hbm,
                    sel_hbm, tgt_hbm, ps_hbm, cs_hbm, cb_hbm,
                    xv, pv, parv, psv, csv, cbv, selv, tgtv):
    wid = lax.axis_index("s") * NC + lax.axis_index("c")
    tb = wid * CHUNK

    pltpu.sync_copy(x_hbm.at[pl.ds(tb * D, CHUNK * D)], xv)
    pltpu.sync_copy(pos_hbm.at[pl.ds(tb, CHUNK)], pv)
    pltpu.sync_copy(par_hbm, parv)

    def prow(i):
        return parv[pl.ds(i * L, L)]

    thr = prow(0)
    pw = prow(1)
    cw = prow(2)
    pearly = prow(3)
    plate = prow(4)

    lane = lax.iota(jnp.int32, L)

    wsc = []
    for t in range(T):
        wt = prow(5 + t)
        wsc.append([wt[c] for c in range(D)])

    esp, lsp = [], []
    for t in range(T):
        wt = prow(5 + T + t)
        esp.append(jnp.full((L,), jnp.sum(wt * pearly)))
        lsp.append(jnp.full((L,), jnp.sum(wt * plate)))

    def block(blk, carry):
        base = blk * L
        pvec = pv[pl.ds(base, L)]
        mask = pvec.astype(jnp.float32) < thr

        gidx = lane * D + blk * (L * D)
        xT = [plsc.load_gather(xv, [gidx + c]) for c in range(D)]
        xR = [_bf16_round(v) for v in xT]

        sidx_base = lane * T + blk * (L * T)
        best = None
        bidx = None
        for t in range(T):
            acc = xR[0] * wsc[t][0]
            for c in range(1, D):
                acc = acc + xR[c] * wsc[t][c]
            post = jnp.where(mask, esp[t], lsp[t])
            comb = pw * post + cw * acc
            sidx = sidx_base + t
            plsc.store_scatter(psv, [sidx], post)
            plsc.store_scatter(csv, [sidx], acc)
            plsc.store_scatter(cbv, [sidx], comb)
            if t == 0:
                best, bidx = comb, _full(0)
            else:
                gt = comb > best
                best = jnp.where(gt, comb, best)
                bidx = jnp.where(gt, _full(t), bidx)

        pos_class = jnp.where(mask, _full(0), _full(1))
        f0 = (xT[0] > 0).astype(jnp.int32)
        f1 = (xT[1] > 0).astype(jnp.int32)
        selv[pl.ds(base, L)] = bidx
        tgtv[pl.ds(base, L)] = pos_class * 4 + f0 * 2 + f1
        return carry

    lax.fori_loop(0, BLOCKS, block, 0)

    pltpu.sync_copy(selv, sel_hbm.at[pl.ds(tb, CHUNK)])
    pltpu.sync_copy(tgtv, tgt_hbm.at[pl.ds(tb, CHUNK)])
    pltpu.sync_copy(psv, ps_hbm.at[pl.ds(tb * T, CHUNK * T)])
    pltpu.sync_copy(csv, cs_hbm.at[pl.ds(tb * T, CHUNK * T)])
    pltpu.sync_copy(cbv, cb_hbm.at[pl.ds(tb * T, CHUNK * T)])


_OUT_TYPE = (
    jax.ShapeDtypeStruct((N_TOK,), jnp.int32),
    jax.ShapeDtypeStruct((N_TOK,), jnp.int32),
    jax.ShapeDtypeStruct((N_TOK * T,), jnp.float32),
    jax.ShapeDtypeStruct((N_TOK * T,), jnp.float32),
    jax.ShapeDtypeStruct((N_TOK * T,), jnp.float32),
)

_SCRATCH = (
    pltpu.VMEM((CHUNK * D,), jnp.float32),
    pltpu.VMEM((CHUNK,), jnp.int32),
    pltpu.VMEM((NPAR * L,), jnp.float32),
    pltpu.VMEM((CHUNK * T,), jnp.float32),
    pltpu.VMEM((CHUNK * T,), jnp.float32),
    pltpu.VMEM((CHUNK * T,), jnp.float32),
    pltpu.VMEM((CHUNK,), jnp.int32),
    pltpu.VMEM((CHUNK,), jnp.int32),
)

@functools.lru_cache(maxsize=None)
def _sc_router():
    return pl.kernel(
        _sc_router_body,
        out_type=_OUT_TYPE,
        mesh=plsc.VectorSubcoreMesh(core_axis_name="c", subcore_axis_name="s",
                                    num_cores=NC, num_subcores=NS),
        scratch_types=_SCRATCH,
        compiler_params=pltpu.CompilerParams(needs_layout_passes=False),
    )


def _b16(v):
    y = lax.bitcast_convert_type(v, jnp.int32)
    odd = lax.shift_right_logical(y, 16) & 1
    r = (y + 32767 + odd) & (-65536)
    return lax.bitcast_convert_type(r, jnp.float32)


def kernel(x, positions, seq_len, position_sigs, content_sigs,
           position_logit, content_logit, pos_early, pos_late):
    B, S, _ = x.shape
    n = B * S
    xf = x.reshape(n * D).astype(jnp.float32)
    pf = positions.reshape(n).astype(jnp.int32)
    half = jnp.asarray(seq_len, jnp.float32) / 2.0
    sp = jax.nn.sigmoid(jnp.asarray(position_logit, jnp.float32))
    sc = jax.nn.sigmoid(jnp.asarray(content_logit, jnp.float32))
    params = jnp.concatenate([
        jnp.full((L,), half, jnp.float32),
        jnp.full((L,), sp / (sp + sc), jnp.float32),
        jnp.full((L,), sc / (sp + sc), jnp.float32),
        _b16(pos_early.astype(jnp.float32)),
        _b16(pos_late.astype(jnp.float32)),
        _b16(jnp.tanh(content_sigs.astype(jnp.float32))).reshape(-1),
        _b16(jnp.tanh(position_sigs.astype(jnp.float32))).reshape(-1),
    ])
    sel, tgt, ps, cs, cb = _sc_router()(xf, pf, params)
    return (sel.reshape(B, S), tgt.reshape(B, S),
            ps.reshape(B, S, T), cs.reshape(B, S, T), cb.reshape(B, S, T))


# speedup vs baseline: 2.1758x; 2.1758x over previous
import functools

import jax
import jax.numpy as jnp
from jax import lax
from jax.experimental import pallas as pl
from jax.experimental.pallas import tpu as pltpu
from jax.experimental.pallas import tpu_sc as plsc

L = 16
NC = 2
NS = 16
NW = NC * NS
T = 8
D = 16
B = 4
S = 8192
N_TOK = B * S
BLK = 128
NBLK = N_TOK // BLK
BPW = NBLK // NW
NG = BLK // L
ST = S // BLK
NPAR = 5 + 2 * T


def _full(v, dtype=jnp.int32):
    return jnp.full((L,), v, dtype)


def _bf16_round(v):
    y = plsc.bitcast(v, jnp.int32)
    odd = lax.shift_right_logical(y, 16) & 1
    r = (y + 32767 + odd) & (-65536)
    return plsc.bitcast(r, jnp.float32)


def _sc_router_body(x_hbm, pos_hbm, par_hbm,
                    sel_hbm, tgt_hbm, ps_hbm, cs_hbm, cb_hbm,
                    xv0, xv1, pv, parv, psv, csv, cbv, selv, tgtv):
    wid = lax.axis_index("s") * NC + lax.axis_index("c")

    pltpu.sync_copy(par_hbm, parv)

    def prow(i):
        return parv[pl.ds(i * L, L)]

    thr = prow(0)
    pw = prow(1)
    cw = prow(2)
    pearly = prow(3)
    plate = prow(4)

    wsc = []
    for t in range(T):
        wt = prow(5 + t)
        wsc.append([wt[c] for c in range(D)])

    esp, lsp = [], []
    for t in range(T):
        wt = prow(5 + T + t)
        esp.append(jnp.full((L,), jnp.sum(wt * pearly)))
        lsp.append(jnp.full((L,), jnp.sum(wt * plate)))

    def block(i, carry):
        blk = wid * BPW + i
        b = blk // ST
        st = blk % ST
        pltpu.sync_copy(x_hbm.at[pl.ds((b * 2 * ST + st) * (8 * BLK), 8 * BLK)], xv0)
        pltpu.sync_copy(x_hbm.at[pl.ds(((b * 2 + 1) * ST + st) * (8 * BLK), 8 * BLK)], xv1)
        pltpu.sync_copy(pos_hbm.at[pl.ds((st * B + b) * BLK, BLK)], pv)

        for q in range(NG):
            pvec = pv[pl.ds(q * L, L)]
            mask = pvec.astype(jnp.float32) < thr

            xT0 = xv0[pl.ds(q * L, L)]
            xT1 = xv0[pl.ds(BLK + q * L, L)]
            xR = []
            for c in range(D):
                src = xv0 if c < 8 else xv1
                xR.append(_bf16_round(src[pl.ds((c % 8) * BLK + q * L, L)]))

            best = None
            bidx = None
            for t in range(T):
                acc = xR[0] * wsc[t][0]
                for c in range(1, D):
                    acc = acc + xR[c] * wsc[t][c]
                post = jnp.where(mask, esp[t], lsp[t])
                comb = pw * post + cw * acc
                psv[pl.ds(t * BLK + q * L, L)] = post
                csv[pl.ds(t * BLK + q * L, L)] = acc
                cbv[pl.ds(t * BLK + q * L, L)] = comb
                if t == 0:
                    best, bidx = comb, _full(0)
                else:
                    gt = comb > best
                    best = jnp.where(gt, comb, best)
                    bidx = jnp.where(gt, _full(t), bidx)

            pos_class = jnp.where(mask, _full(0), _full(1))
            f0 = (xT0 > 0).astype(jnp.int32)
            f1 = (xT1 > 0).astype(jnp.int32)
            selv[pl.ds(q * L, L)] = bidx
            tgtv[pl.ds(q * L, L)] = pos_class * 4 + f0 * 2 + f1

        pltpu.sync_copy(selv, sel_hbm.at[pl.ds((st * B + b) * BLK, BLK)])
        pltpu.sync_copy(tgtv, tgt_hbm.at[pl.ds((st * B + b) * BLK, BLK)])
        pltpu.sync_copy(psv, ps_hbm.at[pl.ds(blk * (T * BLK), T * BLK)])
        pltpu.sync_copy(csv, cs_hbm.at[pl.ds(blk * (T * BLK), T * BLK)])
        pltpu.sync_copy(cbv, cb_hbm.at[pl.ds(blk * (T * BLK), T * BLK)])
        return carry

    lax.fori_loop(0, BPW, block, 0)


_OUT_TYPE = (
    jax.ShapeDtypeStruct((N_TOK,), jnp.int32),
    jax.ShapeDtypeStruct((N_TOK,), jnp.int32),
    jax.ShapeDtypeStruct((N_TOK * T,), jnp.float32),
    jax.ShapeDtypeStruct((N_TOK * T,), jnp.float32),
    jax.ShapeDtypeStruct((N_TOK * T,), jnp.float32),
)

_SCRATCH = (
    pltpu.VMEM((8 * BLK,), jnp.float32),
    pltpu.VMEM((8 * BLK,), jnp.float32),
    pltpu.VMEM((BLK,), jnp.int32),
    pltpu.VMEM((NPAR * L,), jnp.float32),
    pltpu.VMEM((T * BLK,), jnp.float32),
    pltpu.VMEM((T * BLK,), jnp.float32),
    pltpu.VMEM((T * BLK,), jnp.float32),
    pltpu.VMEM((BLK,), jnp.int32),
    pltpu.VMEM((BLK,), jnp.int32),
)


@functools.lru_cache(maxsize=None)
def _sc_router():
    return pl.kernel(
        _sc_router_body,
        out_type=_OUT_TYPE,
        mesh=plsc.VectorSubcoreMesh(core_axis_name="c", subcore_axis_name="s",
                                    num_cores=NC, num_subcores=NS),
        scratch_types=_SCRATCH,
        compiler_params=pltpu.CompilerParams(needs_layout_passes=False),
    )


def _b16(v):
    y = lax.bitcast_convert_type(v, jnp.int32)
    odd = lax.shift_right_logical(y, 16) & 1
    r = (y + 32767 + odd) & (-65536)
    return lax.bitcast_convert_type(r, jnp.float32)


def kernel(x, positions, seq_len, position_sigs, content_sigs,
           position_logit, content_logit, pos_early, pos_late):
    xf = (x.astype(jnp.float32)
          .transpose(0, 2, 1)
          .reshape(B, 2, 8, ST, BLK)
          .transpose(0, 1, 3, 2, 4)
          .reshape(N_TOK * D))
    pf = (positions.astype(jnp.int32)
          .reshape(B, ST, BLK)
          .transpose(1, 0, 2)
          .reshape(N_TOK))
    half = jnp.asarray(seq_len, jnp.float32) / 2.0
    sp = jax.nn.sigmoid(jnp.asarray(position_logit, jnp.float32))
    sc = jax.nn.sigmoid(jnp.asarray(content_logit, jnp.float32))
    params = jnp.concatenate([
        jnp.full((L,), half, jnp.float32),
        jnp.full((L,), sp / (sp + sc), jnp.float32),
        jnp.full((L,), sc / (sp + sc), jnp.float32),
        _b16(pos_early.astype(jnp.float32)),
        _b16(pos_late.astype(jnp.float32)),
        _b16(jnp.tanh(content_sigs.astype(jnp.float32))).reshape(-1),
        _b16(jnp.tanh(position_sigs.astype(jnp.float32))).reshape(-1),
    ])
    sel, tgt, ps, cs, cb = _sc_router()(xf, pf, params)

    def untile_tok(v):
        return v.reshape(ST, B, BLK).transpose(1, 0, 2).reshape(B, S)

    def untile_scores(v):
        return (v.reshape(B, ST, T, BLK)
                .transpose(0, 1, 3, 2)
                .reshape(B, S, T))

    return (untile_tok(sel), untile_tok(tgt),
            untile_scores(ps), untile_scores(cs), untile_scores(cb))


# speedup vs baseline: 3.3369x; 1.5336x over previous
import functools

import jax
import jax.numpy as jnp
from jax import lax
from jax.experimental import pallas as pl
from jax.experimental.pallas import tpu as pltpu
from jax.experimental.pallas import tpu_sc as plsc

L = 16
NC = 2
NS = 16
NW = NC * NS
T = 8
D = 16
B = 4
S = 8192
N_TOK = B * S
BLK = 128
NBLK = N_TOK // BLK
BPW = NBLK // NW
NG = BLK // L
ST = S // BLK
NPAR = 5 + 2 * T


def _full(v, dtype=jnp.int32):
    return jnp.full((L,), v, dtype)


def _bf16_round(v):
    t = v * 65537.0
    return t - (t - v)


WSPLIT = 4


def _sc_router_body(x_hbm, pos_hbm, par_hbm,
                    sel_hbm, tgt_hbm, ps_hbm, cs_hbm, cb_hbm,
                    xv0, xv1, pv, parv, wsp, psv, csv, cbv, selv, tgtv, sem):
    wid = lax.axis_index("s") * NC + lax.axis_index("c")
    b = wid // (ST // BPW)
    st0 = (wid % (ST // BPW)) * BPW

    cps = [
        pltpu.async_copy(par_hbm, parv, sem),
        pltpu.async_copy(x_hbm.at[pl.ds((b * 2 * ST + st0) * (8 * BLK),
                                        BPW * 8 * BLK)], xv0, sem),
        pltpu.async_copy(x_hbm.at[pl.ds(((b * 2 + 1) * ST + st0) * (8 * BLK),
                                        BPW * 8 * BLK)], xv1, sem),
    ]
    for i in range(BPW):
        cps.append(pltpu.async_copy(
            pos_hbm.at[pl.ds(((st0 + i) * B + b) * BLK, BLK)],
            pv.at[pl.ds(i * BLK, BLK)], sem))
    for cp in cps:
        cp.wait()

    def prow(i):
        return parv[pl.ds(i * L, L)]

    thr = prow(0)
    pw = prow(1)
    cw = prow(2)
    pearly = prow(3)
    plate = prow(4)

    wsc = []
    for t in range(T):
        wt = prow(5 + t)
        wsc.append([wt[c] for c in range(D)])
    for t in range(WSPLIT):
        for c in range(D):
            wsp[pl.ds((t * D + c) * L, L)] = jnp.full((L,), wsc[t][c])

    esp, lsp = [], []
    for t in range(T):
        wt = prow(5 + T + t)
        esp.append(jnp.full((L,), jnp.sum(wt * pearly)))
        lsp.append(jnp.full((L,), jnp.sum(wt * plate)))

    def group(g, carry):
        i = g // NG
        q = g % NG
        xb = i * (8 * BLK) + q * L
        pvec = pv[pl.ds(i * BLK + q * L, L)]
        mask = pvec.astype(jnp.float32) < thr

        xT0 = xv0[pl.ds(xb, L)]
        xT1 = xv0[pl.ds(xb + BLK, L)]
        xR = []
        for c in range(D):
            src = xv0 if c < 8 else xv1
            xR.append(_bf16_round(src[pl.ds(xb + (c % 8) * BLK, L)]))

        ob = i * (T * BLK) + q * L
        best = None
        bidx = None
        for t in range(T):
            if t < WSPLIT:
                acc = xR[0] * wsp[pl.ds((t * D) * L, L)]
                for c in range(1, D):
                    acc = acc + xR[c] * wsp[pl.ds((t * D + c) * L, L)]
            else:
                acc = xR[0] * wsc[t][0]
                for c in range(1, D):
                    acc = acc + xR[c] * wsc[t][c]
            post = jnp.where(mask, esp[t], lsp[t])
            comb = pw * post + cw * acc
            psv[pl.ds(ob + t * BLK, L)] = post
            csv[pl.ds(ob + t * BLK, L)] = acc
            cbv[pl.ds(ob + t * BLK, L)] = comb
            if t == 0:
                best, bidx = comb, _full(0)
            else:
                gt = comb > best
                best = jnp.where(gt, comb, best)
                bidx = jnp.where(gt, _full(t), bidx)

        pos_class = jnp.where(mask, _full(0), _full(1))
        f0 = (xT0 > 0).astype(jnp.int32)
        f1 = (xT1 > 0).astype(jnp.int32)
        selv[pl.ds(i * BLK + q * L, L)] = bidx
        tgtv[pl.ds(i * BLK + q * L, L)] = pos_class * 4 + f0 * 2 + f1
        return carry

    lax.fori_loop(0, BPW * NG, group, 0)

    cps = [
        pltpu.async_copy(psv, ps_hbm.at[pl.ds(wid * BPW * T * BLK,
                                              BPW * T * BLK)], sem),
        pltpu.async_copy(csv, cs_hbm.at[pl.ds(wid * BPW * T * BLK,
                                              BPW * T * BLK)], sem),
        pltpu.async_copy(cbv, cb_hbm.at[pl.ds(wid * BPW * T * BLK,
                                              BPW * T * BLK)], sem),
    ]
    for i in range(BPW):
        off = ((st0 + i) * B + b) * BLK
        cps.append(pltpu.async_copy(selv.at[pl.ds(i * BLK, BLK)],
                                    sel_hbm.at[pl.ds(off, BLK)], sem))
        cps.append(pltpu.async_copy(tgtv.at[pl.ds(i * BLK, BLK)],
                                    tgt_hbm.at[pl.ds(off, BLK)], sem))
    for cp in cps:
        cp.wait()


_OUT_TYPE = (
    jax.ShapeDtypeStruct((N_TOK,), jnp.int32),
    jax.ShapeDtypeStruct((N_TOK,), jnp.int32),
    jax.ShapeDtypeStruct((N_TOK * T,), jnp.float32),
    jax.ShapeDtypeStruct((N_TOK * T,), jnp.float32),
    jax.ShapeDtypeStruct((N_TOK * T,), jnp.float32),
)

_SCRATCH = (
    pltpu.VMEM((BPW * 8 * BLK,), jnp.float32),
    pltpu.VMEM((BPW * 8 * BLK,), jnp.float32),
    pltpu.VMEM((BPW * BLK,), jnp.int32),
    pltpu.VMEM((NPAR * L,), jnp.float32),
    pltpu.VMEM((WSPLIT * D * L,), jnp.float32),
    pltpu.VMEM((BPW * T * BLK,), jnp.float32),
    pltpu.VMEM((BPW * T * BLK,), jnp.float32),
    pltpu.VMEM((BPW * T * BLK,), jnp.float32),
    pltpu.VMEM((BPW * BLK,), jnp.int32),
    pltpu.VMEM((BPW * BLK,), jnp.int32),
    pltpu.SemaphoreType.DMA,
)


@functools.lru_cache(maxsize=None)
def _sc_router():
    return pl.kernel(
        _sc_router_body,
        out_type=_OUT_TYPE,
        mesh=plsc.VectorSubcoreMesh(core_axis_name="c", subcore_axis_name="s",
                                    num_cores=NC, num_subcores=NS),
        scratch_types=_SCRATCH,
        compiler_params=pltpu.CompilerParams(needs_layout_passes=False),
    )


def _b16(v):
    y = lax.bitcast_convert_type(v, jnp.int32)
    odd = lax.shift_right_logical(y, 16) & 1
    r = (y + 32767 + odd) & (-65536)
    return lax.bitcast_convert_type(r, jnp.float32)


def kernel(x, positions, seq_len, position_sigs, content_sigs,
           position_logit, content_logit, pos_early, pos_late):
    xf = (x.astype(jnp.float32)
          .transpose(0, 2, 1)
          .reshape(B, 2, 8, ST, BLK)
          .transpose(0, 1, 3, 2, 4)
          .reshape(N_TOK * D))
    pf = (positions.astype(jnp.int32)
          .reshape(B, ST, BLK)
          .transpose(1, 0, 2)
          .reshape(N_TOK))
    half = jnp.asarray(seq_len, jnp.float32) / 2.0
    sp = jax.nn.sigmoid(jnp.asarray(position_logit, jnp.float32))
    sc = jax.nn.sigmoid(jnp.asarray(content_logit, jnp.float32))
    params = jnp.concatenate([
        jnp.full((L,), half, jnp.float32),
        jnp.full((L,), sp / (sp + sc), jnp.float32),
        jnp.full((L,), sc / (sp + sc), jnp.float32),
        _b16(pos_early.astype(jnp.float32)),
        _b16(pos_late.astype(jnp.float32)),
        _b16(jnp.tanh(content_sigs.astype(jnp.float32))).reshape(-1),
        _b16(jnp.tanh(position_sigs.astype(jnp.float32))).reshape(-1),
    ])
    sel, tgt, ps, cs, cb = _sc_router()(xf, pf, params)

    def untile_tok(v):
        return v.reshape(ST, B, BLK).transpose(1, 0, 2).reshape(B, S)

    def untile_scores(v):
        return (v.reshape(B, ST, T, BLK)
                .transpose(0, 1, 3, 2)
                .reshape(B, S, T))

    return (untile_tok(sel), untile_tok(tgt),
            untile_scores(ps), untile_scores(cs), untile_scores(cb))
